# baseline (speedup 1.0000x reference)
import functools

import jax
import jax.numpy as jnp
from jax import lax
from jax.experimental import pallas as pl
from jax.experimental.pallas import tpu as pltpu
from jax.experimental.pallas import tpu_sc as plsc

N_NODES = 10000
D = 128
NC = 2
NS = 16
NW = NC * NS
CHUNK = 128
NBUF = 4
N_ACC = 10240
PAD_DST = N_NODES + 8
SLC = N_ACC // NS

_mesh = plsc.VectorSubcoreMesh(core_axis_name="c", subcore_axis_name="s")


def _make_deg_kernel(nch):
    LAG = 8

    @functools.partial(
        pl.kernel,
        out_type=[jax.ShapeDtypeStruct((N_ACC,), jnp.float32),
                  jax.ShapeDtypeStruct((N_ACC,), jnp.float32)],
        mesh=_mesh,
        scratch_types=[
            pltpu.VMEM((nch, 2 * ECH), jnp.int32),
            pltpu.VMEM((ECH,), jnp.float32),
            pltpu.VMEM((SLC,), jnp.float32),
            pltpu.VMEM_SHARED((N_ACC,), jnp.float32),
            pltpu.SemaphoreType.DMA,
        ],
    )
    def deg_kernel(sd_hbm, out0_hbm, out1_hbm, didx2, ones_v, zbuf, sdeg,
                   sem):
        c = lax.axis_index("c")
        s = lax.axis_index("s")
        wid = c * NS + s

        def _zero(i, _):
            zbuf[pl.ds(i * 16, 16)] = jnp.zeros((16,), jnp.float32)
            return 0
        lax.fori_loop(0, SLC // 16, _zero, 0)
        for j in range(ECH // 16):
            ones_v[pl.ds(j * 16, 16)] = jnp.ones((16,), jnp.float32)
        pltpu.sync_copy(sd_hbm.at[pl.ds(wid * nch, nch)], didx2)
        pltpu.sync_copy(zbuf, sdeg.at[pl.ds(s * SLC, SLC)])
        plsc.subcore_barrier()

        def _fire(i, _):
            pltpu.async_copy(ones_v, sdeg.at[didx2.at[i, pl.ds(ECH, ECH)]],
                             sem, add=True)

            @pl.when(i >= LAG)
            def _():
                pltpu.make_async_copy(
                    ones_v, sdeg.at[didx2.at[0, pl.ds(ECH, ECH)]],
                    sem).wait()
            return 0
        lax.fori_loop(0, nch, _fire, 0)

        def _drain(i, _):
            pltpu.make_async_copy(
                ones_v, sdeg.at[didx2.at[0, pl.ds(ECH, ECH)]], sem).wait()
            return 0
        lax.fori_loop(0, min(LAG, nch), _drain, 0)
        plsc.subcore_barrier()

        pltpu.sync_copy(sdeg.at[pl.ds(s * SLC, SLC)], zbuf)

        @pl.when(c == 0)
        def _():
            pltpu.sync_copy(zbuf, out0_hbm.at[pl.ds(s * SLC, SLC)])

        @pl.when(c == 1)
        def _():
            pltpu.sync_copy(zbuf, out1_hbm.at[pl.ds(s * SLC, SLC)])

    return deg_kernel


ECH = 64
NB = 3


def _make_scatter_kernel(nch):
    assert nch % NB == 0 and nch % 8 == 0

    @functools.partial(
        pl.kernel,
        out_type=[jax.ShapeDtypeStruct((N_ACC, D), jnp.float32),
                  jax.ShapeDtypeStruct((N_ACC, D), jnp.float32)],
        mesh=_mesh,
        scratch_types=(
            [pltpu.VMEM((nch, 2 * ECH), jnp.int32)]
            + [pltpu.VMEM((ECH, D), jnp.float32)] * NB
            + [pltpu.VMEM_SHARED((N_ACC, D), jnp.float32)]
            + [pltpu.SemaphoreType.DMA] * (2 * NB)
        ),
    )
    def scatter_kernel(g_hbm, sd_hbm, out0_hbm, out1_hbm, slab, *rest):
        rows = list(rest[0:NB])
        acc = rest[NB]
        gsem = list(rest[NB + 1:2 * NB + 1])
        ssem = list(rest[2 * NB + 1:3 * NB + 1])
        r0 = rows[0]
        c = lax.axis_index("c")
        s = lax.axis_index("s")
        wid = c * NS + s

        def _zero(i, _):
            r0[i // 8, pl.ds((i % 8) * 16, 16)] = jnp.zeros((16,),
                                                            jnp.float32)
            return 0
        lax.fori_loop(0, ECH * (D // 16), _zero, 0)

        pltpu.sync_copy(sd_hbm.at[pl.ds(wid * nch, nch)], slab)
        for j in range(SLC // ECH):
            pltpu.sync_copy(r0, acc.at[pl.ds(s * SLC + j * ECH, ECH)])
        plsc.subcore_barrier()

        def _sidx(i):
            return slab.at[i, pl.ds(0, ECH)]

        def _didx(i):
            return slab.at[i, pl.ds(ECH, ECH)]

        def _wait_g(k):
            pltpu.make_async_copy(g_hbm.at[_sidx(0)], rows[k],
                                  gsem[k]).wait()

        def _wait_s(k):
            pltpu.make_async_copy(rows[k], acc.at[_didx(0)],
                                  ssem[k]).wait()

        def _outer(io, _):
            for k in range(NB):
                i = io * NB + k

                @pl.when(i >= NB)
                def _():
                    _wait_s(k)
                pltpu.async_copy(g_hbm.at[_sidx(i)], rows[k], gsem[k])

                @pl.when(i >= 1)
                def _():
                    kp = (k - 1) % NB
                    _wait_g(kp)
                    pltpu.async_copy(rows[kp], acc.at[_didx(i - 1)],
                                     ssem[kp], add=True)
            return 0
        lax.fori_loop(0, nch // NB, _outer, 0)

        kl = (nch - 1) % NB
        _wait_g(kl)
        pltpu.async_copy(rows[kl], acc.at[_didx(nch - 1)],
                         ssem[kl], add=True)
        for k in range(NB):
            _wait_s(k)
        plsc.subcore_barrier()

        for j in range(SLC // ECH):
            off = s * SLC + j * ECH
            pltpu.sync_copy(acc.at[pl.ds(off, ECH)], r0)

            @pl.when(c == 0)
            def _():
                pltpu.sync_copy(r0, out0_hbm.at[pl.ds(off, ECH)])

            @pl.when(c == 1)
            def _():
                pltpu.sync_copy(r0, out1_hbm.at[pl.ds(off, ECH)])

    return scatter_kernel


BLK = 400


def _matmul_body(x_ref, w_ref, degp_ref, g_ref):
    deg = degp_ref[:, 0] + degp_ref[:, 1] + 1.0
    dis = lax.rsqrt(deg)
    h = jnp.dot(x_ref[...], w_ref[...], preferred_element_type=jnp.float32)
    g_ref[...] = h * dis[:, None]


def _matmul(x, w, degp_t):
    return pl.pallas_call(
        _matmul_body,
        grid=(N_NODES // BLK,),
        in_specs=[
            pl.BlockSpec((BLK, D), lambda i: (i, 0)),
            pl.BlockSpec((D, D), lambda i: (0, 0)),
            pl.BlockSpec((BLK, NC), lambda i: (i, 0)),
        ],
        out_specs=pl.BlockSpec((BLK, D), lambda i: (i, 0)),
        out_shape=jax.ShapeDtypeStruct((N_NODES, D), jnp.float32),
    )(x, w, degp_t)


def _final_body(p0_ref, p1_ref, g_ref, degp_ref, b_ref, o_ref):
    deg = degp_ref[:, 0] + degp_ref[:, 1] + 1.0
    dis = lax.rsqrt(deg)
    o_ref[...] = (dis[:, None] * (p0_ref[...] + p1_ref[...] + g_ref[...])
                  + b_ref[...])


def _final(p0, p1, g, degp_t, b2d):
    return pl.pallas_call(
        _final_body,
        grid=(N_NODES // BLK,),
        in_specs=[
            pl.BlockSpec((BLK, D), lambda i: (i, 0)),
            pl.BlockSpec((BLK, D), lambda i: (i, 0)),
            pl.BlockSpec((BLK, D), lambda i: (i, 0)),
            pl.BlockSpec((BLK, NC), lambda i: (i, 0)),
            pl.BlockSpec((1, D), lambda i: (0, 0)),
        ],
        out_specs=pl.BlockSpec((BLK, D), lambda i: (i, 0)),
        out_shape=jax.ShapeDtypeStruct((N_NODES, D), jnp.float32),
    )(p0, p1, g, degp_t, b2d)


def kernel(node_features, adjacency_matrix, W, b):
    src = adjacency_matrix[0].astype(jnp.int32)
    dst = adjacency_matrix[1].astype(jnp.int32)
    n_edges = src.shape[0]
    quantum = NW * ECH * 24
    n_pad = (-n_edges) % quantum
    if n_pad:
        src = jnp.concatenate([src, jnp.zeros((n_pad,), jnp.int32)])
        dst = jnp.concatenate([dst, jnp.full((n_pad,), PAD_DST, jnp.int32)])
    n_tot = n_edges + n_pad
    nch = n_tot // (NW * ECH)
    sd = jnp.concatenate([src.reshape(NW * nch, ECH),
                          dst.reshape(NW * nch, ECH)], axis=1)

    d0, d1 = _make_deg_kernel(nch)(sd)
    degp_t = jnp.stack([d0[:N_NODES], d1[:N_NODES]], axis=1)
    g = _matmul(node_features, W, degp_t)
    p0, p1 = _make_scatter_kernel(nch)(g, sd)
    return _final(p0, p1, g, degp_t, b.reshape(1, D))


# speedup vs baseline: 2.3448x; 2.3448x over previous
import functools

import jax
import jax.numpy as jnp
from jax import lax
from jax.experimental import pallas as pl
from jax.experimental.pallas import tpu as pltpu
from jax.experimental.pallas import tpu_sc as plsc

N_NODES = 10000
D = 128
NC = 2
NS = 16
NW = NC * NS
CHUNK = 128
NBUF = 4
N_ACC = 10240
PAD_DST = N_NODES + 8
SLC = N_ACC // NS

_mesh = plsc.VectorSubcoreMesh(core_axis_name="c", subcore_axis_name="s")


def _make_deg_kernel(nch):
    LAG = 8

    @functools.partial(
        pl.kernel,
        out_type=[jax.ShapeDtypeStruct((N_ACC,), jnp.float32),
                  jax.ShapeDtypeStruct((N_ACC,), jnp.float32)],
        mesh=_mesh,
        scratch_types=[
            pltpu.VMEM((nch, CHUNK), jnp.int32),
            pltpu.VMEM((CHUNK,), jnp.float32),
            pltpu.VMEM((SLC,), jnp.float32),
            pltpu.VMEM_SHARED((N_ACC,), jnp.float32),
            pltpu.SemaphoreType.DMA,
        ],
    )
    def deg_kernel(dst3_hbm, out0_hbm, out1_hbm, didx2, ones_v, zbuf, sdeg,
                   sem):
        c = lax.axis_index("c")
        s = lax.axis_index("s")
        wid = c * NS + s

        def _zero(i, _):
            zbuf[pl.ds(i * 16, 16)] = jnp.zeros((16,), jnp.float32)
            return 0
        lax.fori_loop(0, SLC // 16, _zero, 0)
        for j in range(CHUNK // 16):
            ones_v[pl.ds(j * 16, 16)] = jnp.ones((16,), jnp.float32)
        pltpu.sync_copy(dst3_hbm.at[pl.ds(wid * nch, nch)], didx2)
        pltpu.sync_copy(zbuf, sdeg.at[pl.ds(s * SLC, SLC)])
        plsc.subcore_barrier()

        def _fire(i, _):
            pltpu.async_copy(ones_v, sdeg.at[didx2.at[i]], sem, add=True)

            @pl.when(i >= LAG)
            def _():
                pltpu.make_async_copy(ones_v, sdeg.at[didx2.at[0]],
                                      sem).wait()
            return 0
        lax.fori_loop(0, nch, _fire, 0)

        def _drain(i, _):
            pltpu.make_async_copy(ones_v, sdeg.at[didx2.at[0]], sem).wait()
            return 0
        lax.fori_loop(0, min(LAG, nch), _drain, 0)
        plsc.subcore_barrier()

        pltpu.sync_copy(sdeg.at[pl.ds(s * SLC, SLC)], zbuf)

        @pl.when(c == 0)
        def _():
            pltpu.sync_copy(zbuf, out0_hbm.at[pl.ds(s * SLC, SLC)])

        @pl.when(c == 1)
        def _():
            pltpu.sync_copy(zbuf, out1_hbm.at[pl.ds(s * SLC, SLC)])

    return deg_kernel


ECH = 64
NB = 4
PK = 128 // ECH


def _make_scatter_kernel(nch):
    assert nch % NB == 0 and NB % PK == 0 and NB >= 4

    @functools.partial(
        pl.kernel,
        out_type=[jax.ShapeDtypeStruct((N_ACC, D), jnp.float32),
                  jax.ShapeDtypeStruct((N_ACC, D), jnp.float32)],
        mesh=_mesh,
        scratch_types=(
            [pltpu.VMEM((nch // PK, 128), jnp.int32)]
            + [pltpu.VMEM((ECH,), jnp.int32)] * NB
            + [pltpu.VMEM((ECH, D), jnp.float32)] * NB
            + [pltpu.VMEM_SHARED((N_ACC, D), jnp.float32)]
            + [pltpu.SemaphoreType.DMA] * (3 * NB)
        ),
    )
    def scatter_kernel(g_hbm, src3_hbm, dst3_hbm, out0_hbm, out1_hbm,
                       didx2, *rest):
        sidx = list(rest[0:NB])
        rows = list(rest[NB:2 * NB])
        acc = rest[2 * NB]
        isem = list(rest[2 * NB + 1:3 * NB + 1])
        gsem = list(rest[3 * NB + 1:4 * NB + 1])
        ssem = list(rest[4 * NB + 1:5 * NB + 1])
        r0 = rows[0]
        c = lax.axis_index("c")
        s = lax.axis_index("s")
        wid = c * NS + s
        base = wid * nch

        def _zero(i, _):
            r0[i // 8, pl.ds((i % 8) * 16, 16)] = jnp.zeros((16,),
                                                            jnp.float32)
            return 0
        lax.fori_loop(0, ECH * (D // 16), _zero, 0)

        pltpu.sync_copy(dst3_hbm.at[pl.ds(wid * (nch // PK), nch // PK)],
                        didx2)
        for j in range(SLC // ECH):
            pltpu.sync_copy(r0, acc.at[pl.ds(s * SLC + j * ECH, ECH)])
        plsc.subcore_barrier()

        for k in range(NB):
            pltpu.async_copy(src3_hbm.at[base + k], sidx[k], isem[k])

        LEAD = NB - 2

        def _wait_i(k):
            pltpu.make_async_copy(src3_hbm.at[base], sidx[k], isem[k]).wait()

        def _wait_g(k):
            pltpu.make_async_copy(g_hbm.at[sidx[0]], rows[k], gsem[k]).wait()

        def _didx(i, q):
            return didx2.at[i // PK, pl.ds(q * ECH, ECH)]

        def _wait_s(k):
            pltpu.make_async_copy(rows[k], acc.at[_didx(0, 0)],
                                  ssem[k]).wait()

        def _outer(io, _):
            for k in range(NB):
                i = io * NB + k

                @pl.when(i >= NB)
                def _():
                    _wait_s(k)

                @pl.when((i >= 2) & (i + LEAD < nch))
                def _():
                    pltpu.async_copy(src3_hbm.at[base + i + LEAD],
                                     sidx[(k + LEAD) % NB],
                                     isem[(k + LEAD) % NB])
                _wait_i(k)
                pltpu.async_copy(g_hbm.at[sidx[k]], rows[k], gsem[k])

                @pl.when(i >= 1)
                def _():
                    kp = (k - 1) % NB
                    _wait_g(kp)
                    pltpu.async_copy(rows[kp],
                                     acc.at[_didx(i - 1, (k - 1) % PK)],
                                     ssem[kp], add=True)
            return 0
        lax.fori_loop(0, nch // NB, _outer, 0)

        kl = (nch - 1) % NB
        _wait_g(kl)
        pltpu.async_copy(rows[kl], acc.at[_didx(nch - 1, (nch - 1) % PK)],
                         ssem[kl], add=True)
        for k in range(NB):
            _wait_s(k)
        plsc.subcore_barrier()

        for j in range(SLC // ECH):
            off = s * SLC + j * ECH
            pltpu.sync_copy(acc.at[pl.ds(off, ECH)], r0)

            @pl.when(c == 0)
            def _():
                pltpu.sync_copy(r0, out0_hbm.at[pl.ds(off, ECH)])

            @pl.when(c == 1)
            def _():
                pltpu.sync_copy(r0, out1_hbm.at[pl.ds(off, ECH)])

    return scatter_kernel


BLK = 400


def _matmul_body(x_ref, w_ref, degp_ref, g_ref):
    deg = degp_ref[:, 0] + degp_ref[:, 1] + 1.0
    dis = lax.rsqrt(deg)
    h = jnp.dot(x_ref[...], w_ref[...], preferred_element_type=jnp.float32)
    g_ref[...] = h * dis[:, None]


def _matmul(x, w, degp_t):
    return pl.pallas_call(
        _matmul_body,
        grid=(N_NODES // BLK,),
        in_specs=[
            pl.BlockSpec((BLK, D), lambda i: (i, 0)),
            pl.BlockSpec((D, D), lambda i: (0, 0)),
            pl.BlockSpec((BLK, NC), lambda i: (i, 0)),
        ],
        out_specs=pl.BlockSpec((BLK, D), lambda i: (i, 0)),
        out_shape=jax.ShapeDtypeStruct((N_NODES, D), jnp.float32),
    )(x, w, degp_t)


def _final_body(p0_ref, p1_ref, g_ref, degp_ref, b_ref, o_ref):
    deg = degp_ref[:, 0] + degp_ref[:, 1] + 1.0
    dis = lax.rsqrt(deg)
    o_ref[...] = (dis[:, None] * (p0_ref[...] + p1_ref[...] + g_ref[...])
                  + b_ref[...])


def _final(p0, p1, g, degp_t, b2d):
    return pl.pallas_call(
        _final_body,
        grid=(N_NODES // BLK,),
        in_specs=[
            pl.BlockSpec((BLK, D), lambda i: (i, 0)),
            pl.BlockSpec((BLK, D), lambda i: (i, 0)),
            pl.BlockSpec((BLK, D), lambda i: (i, 0)),
            pl.BlockSpec((BLK, NC), lambda i: (i, 0)),
            pl.BlockSpec((1, D), lambda i: (0, 0)),
        ],
        out_specs=pl.BlockSpec((BLK, D), lambda i: (i, 0)),
        out_shape=jax.ShapeDtypeStruct((N_NODES, D), jnp.float32),
    )(p0, p1, g, degp_t, b2d)


def kernel(node_features, adjacency_matrix, W, b):
    src = adjacency_matrix[0].astype(jnp.int32)
    dst = adjacency_matrix[1].astype(jnp.int32)
    n_edges = src.shape[0]
    quantum = NW * CHUNK * 8
    n_pad = (-n_edges) % quantum
    if n_pad:
        src = jnp.concatenate([src, jnp.zeros((n_pad,), jnp.int32)])
        dst = jnp.concatenate([dst, jnp.full((n_pad,), PAD_DST, jnp.int32)])
    n_tot = n_edges + n_pad
    nch = n_tot // (NW * CHUNK)
    nch_sc = n_tot // (NW * ECH)
    dst3 = dst.reshape(NW * nch, CHUNK)
    src4 = src.reshape(NW * nch_sc, ECH)
    dst4 = dst.reshape(NW * nch_sc // PK, PK * ECH)

    d0, d1 = _make_deg_kernel(nch)(dst3)
    degp_t = jnp.stack([d0[:N_NODES], d1[:N_NODES]], axis=1)
    g = _matmul(node_features, W, degp_t)
    p0, p1 = _make_scatter_kernel(nch_sc)(g, src4, dst4)
    return _final(p0, p1, g, degp_t, b.reshape(1, D))
